# Initial kernel scaffold; baseline (speedup 1.0000x reference)
import jax
import jax.numpy as jnp
from jax.experimental import pallas as pl


def kernel(feat, edge_index, gate_w, gate_b, bias):
    raise NotImplementedError("write your pallas kernel here")


# speedup vs baseline: 2.7985x; 2.7985x over previous
import functools

import jax
import jax.numpy as jnp
from jax import lax
from jax.experimental import pallas as pl
from jax.experimental.pallas import tpu as pltpu
from jax.experimental.pallas import tpu_sc as plsc

N_NODES = 10000
N_EDGES = 160000
D = 256
DH = 128

NC, NS = 2, 16
NW = NC * NS
NP = 10240
NT = 10016
DUMMY = N_NODES
EPW = 5120
EP = NW * EPW
CH = 64
NCH = EPW // CH
ROWS_PT = NP // NS


def _tc_pre(feat_p, wd, ws, gb):
    def body(feat_ref, wd_ref, ws_ref, gb_ref, ab_ref):
        x = feat_ref[...]
        a = jnp.sum(x * wd_ref[0, :][None, :], axis=1) + gb_ref[0, 0]
        b = jnp.sum(x * ws_ref[0, :][None, :], axis=1)
        ab_ref[0:1, :] = a[None, :]
        ab_ref[1:2, :] = b[None, :]

    grid = 10
    blk = NP // grid
    return pl.pallas_call(
        body,
        grid=(grid,),
        in_specs=[
            pl.BlockSpec((blk, D), lambda i: (i, 0)),
            pl.BlockSpec((1, D), lambda i: (0, 0)),
            pl.BlockSpec((1, D), lambda i: (0, 0)),
            pl.BlockSpec((1, 1), lambda i: (0, 0)),
        ],
        out_specs=pl.BlockSpec((8, blk), lambda i: (0, i)),
        out_shape=jax.ShapeDtypeStruct((8, NP), jnp.float32),
    )(feat_p, wd, ws, gb)


def _sc_main(fh0, fh1, a1, b1, src3, dst3):
    mesh = plsc.VectorSubcoreMesh(core_axis_name="c", subcore_axis_name="s")

    @functools.partial(
        pl.kernel,
        mesh=mesh,
        out_type=(jax.ShapeDtypeStruct((NC, 2, NP, DH), jnp.float32),
                  jax.ShapeDtypeStruct((NC, NS, NT), jnp.float32)),
        scratch_types=[
            pltpu.VMEM_SHARED((NP, DH), jnp.float32),
            pltpu.VMEM((NT,), jnp.float32),
            pltpu.VMEM((NT,), jnp.float32),
            pltpu.VMEM((NCH, CH), jnp.int32),
            pltpu.VMEM((NCH, CH), jnp.int32),
            pltpu.VMEM((CH, DH), jnp.float32),
            pltpu.SemaphoreType.DMA,
        ],
        compiler_params=pltpu.CompilerParams(needs_layout_passes=False),
    )
    def body(fh0_r, fh1_r, a_r, b_r, src_r, dst_r, out_r, deg_r,
             acc, a_v, b_v, src_v, dst_v, rows_v, sem):
        cid = lax.axis_index("c")
        sid = lax.axis_index("s")
        wid = cid * NS + sid

        pltpu.sync_copy(a_r.at[pl.ds(0, NT)], a_v)
        pltpu.sync_copy(b_r.at[pl.ds(0, NT)], b_v)
        pltpu.sync_copy(src_r.at[wid], src_v)
        pltpu.sync_copy(dst_r.at[wid], dst_v)

        zeros16 = jnp.zeros((16,), jnp.float32)
        ones16 = jnp.ones((16,), jnp.float32)

        for p in range(2):
            fh = fh0_r if p == 0 else fh1_r

            def zero_rows(r, _):
                for q in range(DH // 16):
                    rows_v[r, pl.ds(q * 16, 16)] = zeros16
                return 0

            lax.fori_loop(0, CH, zero_rows, 0)

            def zero_chunk(z, _):
                pltpu.sync_copy(rows_v, acc.at[pl.ds(sid * ROWS_PT + z * CH, CH)])
                return 0

            lax.fori_loop(0, ROWS_PT // CH, zero_chunk, 0)
            plsc.subcore_barrier()

            def chunk(c, _):
                pltpu.async_copy(fh.at[src_v.at[c]], rows_v, sem).wait()

                def group(g, _):
                    d_idx = dst_v[c, pl.ds(g * 16, 16)]
                    s_idx = src_v[c, pl.ds(g * 16, 16)]
                    x = (plsc.load_gather(a_v, [d_idx])
                         + plsc.load_gather(b_v, [s_idx]))
                    t = jnp.exp(-2.0 * jnp.abs(x))
                    s16 = jnp.sign(x) * (1.0 - t) / (1.0 + t)
                    for rr in range(16):
                        r = g * 16 + rr
                        s = s16[rr]
                        for q in range(DH // 16):
                            rows_v[r, pl.ds(q * 16, 16)] = (
                                rows_v[r, pl.ds(q * 16, 16)] * s)
                    return 0

                lax.fori_loop(0, CH // 16, group, 0)
                pltpu.sync_copy(rows_v, acc.at[dst_v.at[c]], add=True)
                return 0

            lax.fori_loop(0, NCH, chunk, 0)
            plsc.subcore_barrier()
            pltpu.sync_copy(acc.at[pl.ds(sid * ROWS_PT, ROWS_PT)],
                            out_r.at[cid, p, pl.ds(sid * ROWS_PT, ROWS_PT)])

        def zero_deg(j, _):
            a_v[pl.ds(j * 16, 16)] = zeros16
            return 0

        lax.fori_loop(0, NT // 16, zero_deg, 0)

        def deg_chunk(c, _):
            for v in range(CH // 16):
                d_idx = dst_v[c, pl.ds(v * 16, 16)]
                plsc.addupdate_scatter(a_v, [d_idx], ones16)
            return 0

        lax.fori_loop(0, NCH, deg_chunk, 0)
        pltpu.sync_copy(a_v, deg_r.at[cid, sid])

    return body(fh0, fh1, a1, b1, src3, dst3)


def _tc_post(psum, degs, feat_p, bias2):
    def body(ps_ref, degs_ref, feat_ref, bias_ref, out_ref):
        ps = ps_ref[...]
        s0 = ps[0, 0] + ps[1, 0]
        s1 = ps[0, 1] + ps[1, 1]
        deg = jnp.sum(degs_ref[...], axis=(0, 1))
        summ = jnp.concatenate([s0, s1], axis=1)
        mean = summ / jnp.maximum(deg, 1.0)[:, None]
        out_ref[...] = mean + bias_ref[0, :][None, :] + feat_ref[...]

    grid = 10
    blk = NP // grid
    return pl.pallas_call(
        body,
        grid=(grid,),
        in_specs=[
            pl.BlockSpec((NC, 2, blk, DH), lambda i: (0, 0, i, 0)),
            pl.BlockSpec((NC, NS, blk), lambda i: (0, 0, i)),
            pl.BlockSpec((blk, D), lambda i: (i, 0)),
            pl.BlockSpec((1, D), lambda i: (0, 0)),
        ],
        out_specs=pl.BlockSpec((blk, D), lambda i: (i, 0)),
        out_shape=jax.ShapeDtypeStruct((NP, D), jnp.float32),
    )(psum, degs, feat_p, bias2)


@jax.jit
def kernel(feat, edge_index, gate_w, gate_b, bias):
    src = edge_index[0].astype(jnp.int32)
    dst = edge_index[1].astype(jnp.int32)
    pad = jnp.full((EP - N_EDGES,), DUMMY, jnp.int32)
    src3 = jnp.concatenate([src, pad]).reshape(NW, NCH, CH)
    dst3 = jnp.concatenate([dst, pad]).reshape(NW, NCH, CH)

    feat_p = jnp.pad(feat, ((0, NP - N_NODES), (0, 0)))
    fh0 = feat_p[:, :DH]
    fh1 = feat_p[:, DH:]

    wd = gate_w[:, :D]
    ws = gate_w[:, D:]
    gb = gate_b.reshape(1, 1)

    ab = _tc_pre(feat_p, wd, ws, gb)
    psum, degs = _sc_main(fh0, fh1, ab[0], ab[1], src3, dst3)
    degs_p = jnp.pad(degs, ((0, 0), (0, 0), (0, NP - NT)))
    out_p = _tc_post(psum, degs_p, feat_p, bias.reshape(1, D))
    return out_p[:N_NODES]


# speedup vs baseline: 3.8776x; 1.3856x over previous
import functools

import jax
import jax.numpy as jnp
from jax import lax
from jax.experimental import pallas as pl
from jax.experimental.pallas import tpu as pltpu
from jax.experimental.pallas import tpu_sc as plsc

N_NODES = 10000
N_EDGES = 160000
D = 256
DH = 128

NC, NS = 2, 16
NW = NC * NS
NP = 10240
NT = 10016
DUMMY = N_NODES
EPW = 5120
EP = NW * EPW
CH = 80
NCH = EPW // CH
ROWS_PT = NP // NS


def _tc_pre(feat_p, wd, ws, gb):
    def body(feat_ref, wd_ref, ws_ref, gb_ref, ab_ref):
        x = feat_ref[...]
        a = jnp.sum(x * wd_ref[0, :][None, :], axis=1) + gb_ref[0, 0]
        b = jnp.sum(x * ws_ref[0, :][None, :], axis=1)
        ab_ref[0:1, :] = a[None, :]
        ab_ref[1:2, :] = b[None, :]

    grid = 10
    blk = NP // grid
    return pl.pallas_call(
        body,
        grid=(grid,),
        in_specs=[
            pl.BlockSpec((blk, D), lambda i: (i, 0)),
            pl.BlockSpec((1, D), lambda i: (0, 0)),
            pl.BlockSpec((1, D), lambda i: (0, 0)),
            pl.BlockSpec((1, 1), lambda i: (0, 0)),
        ],
        out_specs=pl.BlockSpec((8, blk), lambda i: (0, i)),
        out_shape=jax.ShapeDtypeStruct((8, NP), jnp.float32),
    )(feat_p, wd, ws, gb)


def _sc_main(fh0, fh1, a1, b1, src3, dst3):
    mesh = plsc.VectorSubcoreMesh(core_axis_name="c", subcore_axis_name="s")

    @functools.partial(
        pl.kernel,
        mesh=mesh,
        out_type=(jax.ShapeDtypeStruct((NC, 2, NP, DH), jnp.float32),
                  jax.ShapeDtypeStruct((NC, NS, NT), jnp.float32)),
        scratch_types=[
            pltpu.VMEM_SHARED((NP, DH), jnp.float32),
            pltpu.VMEM((NCH, CH), jnp.int32),
            pltpu.VMEM((NCH, CH), jnp.int32),
            pltpu.VMEM((EPW,), jnp.float32),
            pltpu.SemaphoreType.DMA,
        ],
        compiler_params=pltpu.CompilerParams(needs_layout_passes=False),
    )
    def body(fh0_r, fh1_r, a_r, b_r, src_r, dst_r, out_r, deg_r,
             acc, src_v, dst_v, sc_v, sem):
        cid = lax.axis_index("c")
        sid = lax.axis_index("s")
        wid = cid * NS + sid

        pltpu.sync_copy(src_r.at[wid], src_v)
        pltpu.sync_copy(dst_r.at[wid], dst_v)

        zeros16 = jnp.zeros((16,), jnp.float32)
        ones16 = jnp.ones((16,), jnp.float32)

        def phase_a(a_v, b_v):
            pltpu.sync_copy(a_r.at[pl.ds(0, NT)], a_v)
            pltpu.sync_copy(b_r.at[pl.ds(0, NT)], b_v)

            def score_chunk(c, _):
                for g in range(CH // 16):
                    d_idx = dst_v[c, pl.ds(g * 16, 16)]
                    s_idx = src_v[c, pl.ds(g * 16, 16)]
                    x = (plsc.load_gather(a_v, [d_idx])
                         + plsc.load_gather(b_v, [s_idx]))
                    t = jnp.exp(-2.0 * jnp.abs(x))
                    s16 = jnp.sign(x) * (1.0 - t) / (1.0 + t)
                    sc_v[pl.ds(c * CH + g * 16, 16)] = s16
                return 0

            lax.fori_loop(0, NCH, score_chunk, 0)

            def zero_deg(j, _):
                a_v[pl.ds(j * 16, 16)] = zeros16
                return 0

            lax.fori_loop(0, NT // 16, zero_deg, 0)

            def deg_chunk(c, _):
                for v in range(CH // 16):
                    d_idx = dst_v[c, pl.ds(v * 16, 16)]
                    plsc.addupdate_scatter(a_v, [d_idx], ones16)
                return 0

            lax.fori_loop(0, NCH, deg_chunk, 0)
            pltpu.sync_copy(a_v, deg_r.at[cid, sid])

        pl.run_scoped(phase_a,
                      pltpu.VMEM((NT,), jnp.float32),
                      pltpu.VMEM((NT,), jnp.float32))

        def phase_b(rows0, rows1):
            bufs = (rows0, rows1)

            def scale_scatter(c, rows_v):
                def group(g, _):
                    s16 = sc_v[pl.ds(c * CH + g * 16, 16)]
                    for rr in range(16):
                        r = g * 16 + rr
                        s = s16[rr]
                        for q in range(DH // 16):
                            rows_v[r, pl.ds(q * 16, 16)] = (
                                rows_v[r, pl.ds(q * 16, 16)] * s)
                    return 0

                lax.fori_loop(0, CH // 16, group, 0)
                pltpu.sync_copy(rows_v, acc.at[dst_v.at[c]], add=True)

            for p in range(2):
                fh = fh0_r if p == 0 else fh1_r

                def zero_rows(r, _):
                    for q in range(DH // 16):
                        rows0[r, pl.ds(q * 16, 16)] = zeros16
                    return 0

                lax.fori_loop(0, CH, zero_rows, 0)

                def zero_chunk(z, _):
                    pltpu.sync_copy(
                        rows0, acc.at[pl.ds(sid * ROWS_PT + z * CH, CH)])
                    return 0

                lax.fori_loop(0, ROWS_PT // CH, zero_chunk, 0)
                plsc.subcore_barrier()

                pltpu.async_copy(fh.at[src_v.at[0]], rows0, sem)

                def pair(cc, _):
                    c0 = 2 * cc
                    pltpu.async_copy(fh.at[src_v.at[c0 + 1]], rows1, sem)
                    pltpu.make_async_copy(fh.at[src_v.at[c0]], rows0, sem).wait()
                    scale_scatter(c0, rows0)

                    @pl.when(cc + 1 < NCH // 2)
                    def _():
                        pltpu.async_copy(fh.at[src_v.at[c0 + 2]], rows0, sem)

                    pltpu.make_async_copy(
                        fh.at[src_v.at[c0 + 1]], rows1, sem).wait()
                    scale_scatter(c0 + 1, rows1)
                    return 0

                lax.fori_loop(0, NCH // 2, pair, 0)
                plsc.subcore_barrier()
                pltpu.sync_copy(acc.at[pl.ds(sid * ROWS_PT, ROWS_PT)],
                                out_r.at[cid, p, pl.ds(sid * ROWS_PT, ROWS_PT)])

        pl.run_scoped(phase_b,
                      pltpu.VMEM((CH, DH), jnp.float32),
                      pltpu.VMEM((CH, DH), jnp.float32))

    return body(fh0, fh1, a1, b1, src3, dst3)


def _tc_post(psum, degs, feat_p, bias2):
    def body(ps_ref, degs_ref, feat_ref, bias_ref, out_ref):
        ps = ps_ref[...]
        s0 = ps[0, 0] + ps[1, 0]
        s1 = ps[0, 1] + ps[1, 1]
        deg = jnp.sum(degs_ref[...], axis=(0, 1))
        summ = jnp.concatenate([s0, s1], axis=1)
        mean = summ / jnp.maximum(deg, 1.0)[:, None]
        out_ref[...] = mean + bias_ref[0, :][None, :] + feat_ref[...]

    grid = 10
    blk = NP // grid
    return pl.pallas_call(
        body,
        grid=(grid,),
        in_specs=[
            pl.BlockSpec((NC, 2, blk, DH), lambda i: (0, 0, i, 0)),
            pl.BlockSpec((NC, NS, blk), lambda i: (0, 0, i)),
            pl.BlockSpec((blk, D), lambda i: (i, 0)),
            pl.BlockSpec((1, D), lambda i: (0, 0)),
        ],
        out_specs=pl.BlockSpec((blk, D), lambda i: (i, 0)),
        out_shape=jax.ShapeDtypeStruct((NP, D), jnp.float32),
    )(psum, degs, feat_p, bias2)


@jax.jit
def kernel(feat, edge_index, gate_w, gate_b, bias):
    src = edge_index[0].astype(jnp.int32)
    dst = edge_index[1].astype(jnp.int32)
    pad = jnp.full((EP - N_EDGES,), DUMMY, jnp.int32)
    src3 = jnp.concatenate([src, pad]).reshape(NW, NCH, CH)
    dst3 = jnp.concatenate([dst, pad]).reshape(NW, NCH, CH)

    feat_p = jnp.pad(feat, ((0, NP - N_NODES), (0, 0)))
    fh0 = feat_p[:, :DH]
    fh1 = feat_p[:, DH:]

    wd = gate_w[:, :D]
    ws = gate_w[:, D:]
    gb = gate_b.reshape(1, 1)

    ab = _tc_pre(feat_p, wd, ws, gb)
    psum, degs = _sc_main(fh0, fh1, ab[0], ab[1], src3, dst3)
    degs_p = jnp.pad(degs, ((0, 0), (0, 0), (0, NP - NT)))
    out_p = _tc_post(psum, degs_p, feat_p, bias.reshape(1, D))
    return out_p[:N_NODES]
